# baseline (speedup 1.0000x reference)
import functools

import jax
import jax.numpy as jnp
from jax import lax
from jax.experimental import pallas as pl
from jax.experimental.pallas import tpu as pltpu
from jax.experimental.pallas import tpu_sc as plsc

SOM_H, SOM_W = 64, 64
N_NODES = SOM_H * SOM_W
LATENT = 256
N_ROWS = 8192
TC_BLOCK = 512

_EPS_F32 = 1.1920929e-07


def _dist_body(z_ref, e_ref, dist_ref, bmu_ref, e2_ref):
    zb = z_ref[...]

    @pl.when(pl.program_id(0) == 0)
    def _():
        eb = e_ref[...]
        e2_ref[...] = jnp.sum(eb * eb, axis=1)[None, :]

    ones_d = jnp.ones((LATENT,), jnp.float32)
    z2 = lax.dot_general(zb * zb, ones_d, (((1,), (0,)), ((), ())),
                         preferred_element_type=jnp.float32)[:, None]
    dot = lax.dot_general(zb, e_ref[...], (((1,), (1,)), ((), ())),
                          preferred_element_type=jnp.float32)
    d = z2 + e2_ref[...] - 2.0 * dot
    d = jnp.maximum(d, 0.0)
    dist_ref[...] = d
    bmu_ref[...] = jnp.argmin(d, axis=1).astype(jnp.int32)


def _q_body(alpha_ref, z_ref, e_ref, q_ref, e2_ref):
    zb = z_ref[...]

    @pl.when(pl.program_id(0) == 0)
    def _():
        eb = e_ref[...]
        e2_ref[...] = jnp.sum(eb * eb, axis=1)[None, :]

    ones_d = jnp.ones((LATENT,), jnp.float32)
    ones_k = jnp.ones((N_NODES,), jnp.float32)
    z2 = lax.dot_general(zb * zb, ones_d, (((1,), (0,)), ((), ())),
                         preferred_element_type=jnp.float32)[:, None]
    dot = lax.dot_general(zb, e_ref[...], (((1,), (1,)), ((), ())),
                          preferred_element_type=jnp.float32)
    d = z2 + e2_ref[...] - 2.0 * dot
    d = jnp.maximum(d, 0.0)

    af = alpha_ref[0, 0]
    ex = (af + 1.0) * 0.5
    ia = 1.0 / af

    def _finish(qn):
        s = lax.dot_general(qn, ones_k, (((1,), (0,)), ((), ())),
                            preferred_element_type=jnp.float32)[:, None]
        q_ref[...] = qn * (1.0 / s) + _EPS_F32

    @pl.when(ex == 1.0)
    def _():
        _finish(1.0 / (1.0 + d * ia))

    @pl.when(ex != 1.0)
    def _():
        _finish(jnp.exp(jnp.log(1.0 / (1.0 + d * ia)) * ex))


def _tc_dist_call(z, e):
    grid = (N_ROWS // TC_BLOCK,)
    return pl.pallas_call(
        _dist_body,
        grid=grid,
        in_specs=[
            pl.BlockSpec((TC_BLOCK, LATENT), lambda i: (i, 0)),
            pl.BlockSpec((N_NODES, LATENT), lambda i: (0, 0)),
        ],
        out_specs=[
            pl.BlockSpec((TC_BLOCK, N_NODES), lambda i: (i, 0)),
            pl.BlockSpec((TC_BLOCK,), lambda i: (i,)),
        ],
        out_shape=[
            jax.ShapeDtypeStruct((N_ROWS, N_NODES), jnp.float32),
            jax.ShapeDtypeStruct((N_ROWS,), jnp.int32),
        ],
        scratch_shapes=[pltpu.VMEM((1, N_NODES), jnp.float32)],
    )(z, e)


def _tc_q_call(af, z, e):
    grid = (N_ROWS // TC_BLOCK,)
    return pl.pallas_call(
        _q_body,
        grid=grid,
        in_specs=[
            pl.BlockSpec(memory_space=pltpu.SMEM),
            pl.BlockSpec((TC_BLOCK, LATENT), lambda i: (i, 0)),
            pl.BlockSpec((N_NODES, LATENT), lambda i: (0, 0)),
        ],
        out_specs=pl.BlockSpec((TC_BLOCK, N_NODES), lambda i: (i, 0)),
        out_shape=jax.ShapeDtypeStruct((N_ROWS, N_NODES), jnp.float32),
        scratch_shapes=[pltpu.VMEM((1, N_NODES), jnp.float32)],
    )(af, z, e)


def _sc_gather(embeddings, bmu):
    info = plsc.get_sparse_core_info()
    nc, ns = info.num_cores, info.num_subcores
    nw = nc * ns
    rows_w = N_ROWS // nw
    zq_chunk = 128
    nbr_rows = 8
    n_zq = rows_w // zq_chunk
    n_nbr = rows_w // nbr_rows

    mesh = plsc.VectorSubcoreMesh(core_axis_name="c", subcore_axis_name="s")

    @functools.partial(
        pl.kernel,
        mesh=mesh,
        out_type=[
            jax.ShapeDtypeStruct((N_ROWS, LATENT), jnp.float32),
            jax.ShapeDtypeStruct((N_ROWS, 8, LATENT), jnp.float32),
        ],
        scratch_types=[
            pltpu.VMEM((rows_w + 8,), jnp.int32),
            [pltpu.VMEM((nbr_rows * 8,), jnp.int32) for _ in range(2)],
            [pltpu.VMEM((zq_chunk, LATENT), jnp.float32) for _ in range(2)],
            [pltpu.VMEM((nbr_rows * 8, LATENT), jnp.float32)
             for _ in range(2)],
            [pltpu.SemaphoreType.DMA for _ in range(4)],
            [pltpu.SemaphoreType.DMA for _ in range(4)],
        ],
    )
    def k(emb_hbm, bmu_hbm, zq_hbm, nbr_hbm, bmu_v, idx_v, zrows_v,
          nrows_v, gsem, wsem):
        wid = lax.axis_index("s") * nc + lax.axis_index("c")
        base = wid * rows_w
        pltpu.sync_copy(bmu_hbm.at[pl.ds(base, rows_w)],
                        bmu_v.at[pl.ds(0, rows_w)])

        lane = lax.iota(jnp.int32, 16)
        dnums = lax.GatherDimensionNumbers(
            offset_dims=(), collapsed_slice_dims=(0,), start_index_map=(0,))

        def nbr_idx(c, idx_ref):
            v = bmu_v[pl.ds(c * nbr_rows, 16)]
            for t in range(nbr_rows * 8 // 16):
                p = t * 16 + lane
                i_rel = lax.shift_right_logical(p, 3)
                j_rel = p & 7
                vi = lax.gather(
                    v, i_rel[:, None], dnums, (1,),
                    mode=lax.GatherScatterMode.PROMISE_IN_BOUNDS)
                k1 = lax.shift_right_logical(vi, 6)
                k2 = vi & 63
                dk1 = (jnp.where(j_rel == 1, -1, 0)
                       + jnp.where(j_rel == 2, 1, 0))
                dk2 = (jnp.where(j_rel == 3, 1, 0)
                       + jnp.where(j_rel == 4, -1, 0))
                idx_ref[pl.ds(t * 16, 16)] = (
                    lax.shift_left((k1 + dk1 + 64) & 63, 6)
                    | ((k2 + dk2 + 64) & 63))

        def zq_out(c):
            return zq_hbm.at[pl.ds(base + c * zq_chunk, zq_chunk)]

        def nbr_out(c):
            return nbr_hbm.at[pl.ds(base + c * nbr_rows, nbr_rows)]

        def nbr_src(b):
            return nrows_v[b].reshape(nbr_rows, 8, LATENT)

        for c in range(n_zq):
            pltpu.async_copy(
                emb_hbm.at[bmu_v.at[pl.ds(c * zq_chunk, zq_chunk)]],
                zrows_v[c], gsem[c])

        nbr_idx(0, idx_v[0])
        pltpu.async_copy(emb_hbm.at[idx_v[0]], nrows_v[0], gsem[2])
        for c in range(n_nbr):
            b = c & 1
            pltpu.make_async_copy(
                emb_hbm.at[idx_v[b]], nrows_v[b], gsem[2 + b]).wait()
            pltpu.async_copy(nbr_src(b), nbr_out(c), wsem[2 + b])
            if c + 1 < n_nbr:
                b2 = (c + 1) & 1
                nbr_idx(c + 1, idx_v[b2])
                if c >= 1:
                    pltpu.make_async_copy(
                        nbr_src(b2), nbr_out(c - 1), wsem[2 + b2]).wait()
                pltpu.async_copy(emb_hbm.at[idx_v[b2]], nrows_v[b2],
                                 gsem[2 + b2])
        for c in (n_nbr - 2, n_nbr - 1):
            pltpu.make_async_copy(
                nbr_src(c & 1), nbr_out(c), wsem[2 + (c & 1)]).wait()

        for c in range(n_zq):
            pltpu.make_async_copy(
                emb_hbm.at[bmu_v.at[pl.ds(c * zq_chunk, zq_chunk)]],
                zrows_v[c], gsem[c]).wait()
            pltpu.async_copy(zrows_v[c], zq_out(c), wsem[c])
        for c in range(n_zq):
            pltpu.make_async_copy(
                zrows_v[c], zq_out(c), wsem[c]).wait()

    return k(embeddings, bmu)


def kernel(z_e_sample, embeddings, alpha_som):
    af = jnp.asarray(alpha_som, jnp.float32).reshape(1, 1)
    z_dist, bmu = _tc_dist_call(z_e_sample, embeddings)
    z_q, nbr8 = _sc_gather(embeddings, bmu)
    q = _tc_q_call(af, z_e_sample, embeddings)
    z_q_neighbors = nbr8[:, :5, :]
    return (z_dist, bmu, z_q, z_q_neighbors, q)


# speedup vs baseline: 1.0498x; 1.0498x over previous
import functools

import jax
import jax.numpy as jnp
from jax import lax
from jax.experimental import pallas as pl
from jax.experimental.pallas import tpu as pltpu
from jax.experimental.pallas import tpu_sc as plsc

SOM_H, SOM_W = 64, 64
N_NODES = SOM_H * SOM_W
LATENT = 256
N_ROWS = 8192
TC_BLOCK = 512

_EPS_F32 = 1.1920929e-07


def _dist_body(z_ref, e_ref, dist_ref, bmu_ref, e2_ref):
    zb = z_ref[...]

    @pl.when(pl.program_id(0) == 0)
    def _():
        eb = e_ref[...]
        e2_ref[...] = jnp.sum(eb * eb, axis=1)[None, :]

    ones_d = jnp.ones((LATENT,), jnp.float32)
    z2 = lax.dot_general(zb * zb, ones_d, (((1,), (0,)), ((), ())),
                         preferred_element_type=jnp.float32)[:, None]
    dot = lax.dot_general(zb, e_ref[...], (((1,), (1,)), ((), ())),
                          preferred_element_type=jnp.float32)
    d = z2 + e2_ref[...] - 2.0 * dot
    d = jnp.maximum(d, 0.0)
    dist_ref[...] = d
    bmu_ref[...] = jnp.argmin(d, axis=1).astype(jnp.int32)


def _q_body(alpha_ref, z_ref, e_ref, q_ref, e2_ref):
    zb = z_ref[...]

    @pl.when(pl.program_id(0) == 0)
    def _():
        eb = e_ref[...]
        e2_ref[...] = jnp.sum(eb * eb, axis=1)[None, :]

    ones_d = jnp.ones((LATENT,), jnp.float32)
    ones_k = jnp.ones((N_NODES,), jnp.float32)
    z2 = lax.dot_general(zb * zb, ones_d, (((1,), (0,)), ((), ())),
                         preferred_element_type=jnp.float32)[:, None]
    dot = lax.dot_general(zb, e_ref[...], (((1,), (1,)), ((), ())),
                          preferred_element_type=jnp.float32)
    d = z2 + e2_ref[...] - 2.0 * dot
    d = jnp.maximum(d, 0.0)

    af = alpha_ref[0, 0]
    ex = (af + 1.0) * 0.5
    ia = 1.0 / af

    def _finish(qn):
        s = lax.dot_general(qn, ones_k, (((1,), (0,)), ((), ())),
                            preferred_element_type=jnp.float32)[:, None]
        q_ref[...] = qn * (1.0 / s) + _EPS_F32

    @pl.when(ex == 1.0)
    def _():
        _finish(1.0 / (1.0 + d * ia))

    @pl.when(ex != 1.0)
    def _():
        _finish(jnp.exp(jnp.log(1.0 / (1.0 + d * ia)) * ex))


def _tc_dist_call(z, e):
    grid = (N_ROWS // TC_BLOCK,)
    return pl.pallas_call(
        _dist_body,
        grid=grid,
        in_specs=[
            pl.BlockSpec((TC_BLOCK, LATENT), lambda i: (i, 0)),
            pl.BlockSpec((N_NODES, LATENT), lambda i: (0, 0)),
        ],
        out_specs=[
            pl.BlockSpec((TC_BLOCK, N_NODES), lambda i: (i, 0)),
            pl.BlockSpec((TC_BLOCK,), lambda i: (i,)),
        ],
        out_shape=[
            jax.ShapeDtypeStruct((N_ROWS, N_NODES), jnp.float32),
            jax.ShapeDtypeStruct((N_ROWS,), jnp.int32),
        ],
        scratch_shapes=[pltpu.VMEM((1, N_NODES), jnp.float32)],
    )(z, e)


def _tc_q_call(af, z, e):
    grid = (N_ROWS // TC_BLOCK,)
    return pl.pallas_call(
        _q_body,
        grid=grid,
        in_specs=[
            pl.BlockSpec(memory_space=pltpu.SMEM),
            pl.BlockSpec((TC_BLOCK, LATENT), lambda i: (i, 0)),
            pl.BlockSpec((N_NODES, LATENT), lambda i: (0, 0)),
        ],
        out_specs=pl.BlockSpec((TC_BLOCK, N_NODES), lambda i: (i, 0)),
        out_shape=jax.ShapeDtypeStruct((N_ROWS, N_NODES), jnp.float32),
        scratch_shapes=[pltpu.VMEM((1, N_NODES), jnp.float32)],
    )(af, z, e)


def _sc_gather(embeddings, bmu):
    info = plsc.get_sparse_core_info()
    nc, ns = info.num_cores, info.num_subcores
    nw = nc * ns
    rows_w = N_ROWS // nw
    zq_chunk = 64
    nbr_rows = 16
    n_zq = rows_w // zq_chunk
    n_nbr = rows_w // nbr_rows

    mesh = plsc.VectorSubcoreMesh(core_axis_name="c", subcore_axis_name="s")

    @functools.partial(
        pl.kernel,
        mesh=mesh,
        out_type=[
            jax.ShapeDtypeStruct((N_ROWS, LATENT), jnp.float32),
            jax.ShapeDtypeStruct((N_ROWS, 8, LATENT), jnp.float32),
        ],
        scratch_types=[
            pltpu.VMEM((rows_w + 8,), jnp.int32),
            [pltpu.VMEM((nbr_rows * 8,), jnp.int32) for _ in range(2)],
            [pltpu.VMEM((zq_chunk, LATENT), jnp.float32) for _ in range(2)],
            [pltpu.VMEM((nbr_rows * 8, LATENT), jnp.float32)
             for _ in range(2)],
            [pltpu.SemaphoreType.DMA for _ in range(4)],
            [pltpu.SemaphoreType.DMA for _ in range(4)],
        ],
    )
    def k(emb_hbm, bmu_hbm, zq_hbm, nbr_hbm, bmu_v, idx_v, zrows_v,
          nrows_v, gsem, wsem):
        wid = lax.axis_index("s") * nc + lax.axis_index("c")
        base = wid * rows_w
        pltpu.sync_copy(bmu_hbm.at[pl.ds(base, rows_w)],
                        bmu_v.at[pl.ds(0, rows_w)])

        lane = lax.iota(jnp.int32, 16)
        dnums = lax.GatherDimensionNumbers(
            offset_dims=(), collapsed_slice_dims=(0,), start_index_map=(0,))

        def nbr_idx(c, idx_ref):
            v = bmu_v[pl.ds(c * nbr_rows, 16)]
            for t in range(nbr_rows * 8 // 16):
                p = t * 16 + lane
                i_rel = lax.shift_right_logical(p, 3)
                j_rel = p & 7
                vi = lax.gather(
                    v, i_rel[:, None], dnums, (1,),
                    mode=lax.GatherScatterMode.PROMISE_IN_BOUNDS)
                k1 = lax.shift_right_logical(vi, 6)
                k2 = vi & 63
                dk1 = (jnp.where(j_rel == 1, -1, 0)
                       + jnp.where(j_rel == 2, 1, 0))
                dk2 = (jnp.where(j_rel == 3, 1, 0)
                       + jnp.where(j_rel == 4, -1, 0))
                idx_ref[pl.ds(t * 16, 16)] = (
                    lax.shift_left((k1 + dk1 + 64) & 63, 6)
                    | ((k2 + dk2 + 64) & 63))

        def zq_out(c):
            return zq_hbm.at[pl.ds(base + c * zq_chunk, zq_chunk)]

        def nbr_out(c):
            return nbr_hbm.at[pl.ds(base + c * nbr_rows, nbr_rows)]

        def nbr_src(b):
            return nrows_v[b].reshape(nbr_rows, 8, LATENT)

        for c in range(2):
            pltpu.async_copy(
                emb_hbm.at[bmu_v.at[pl.ds(c * zq_chunk, zq_chunk)]],
                zrows_v[c], gsem[c])

        nbr_idx(0, idx_v[0])
        pltpu.async_copy(emb_hbm.at[idx_v[0]], nrows_v[0], gsem[2])
        for c in range(n_nbr):
            b = c & 1
            pltpu.make_async_copy(
                emb_hbm.at[idx_v[b]], nrows_v[b], gsem[2 + b]).wait()
            pltpu.async_copy(nbr_src(b), nbr_out(c), wsem[2 + b])
            if c + 1 < n_nbr:
                b2 = (c + 1) & 1
                nbr_idx(c + 1, idx_v[b2])
                if c >= 1:
                    pltpu.make_async_copy(
                        nbr_src(b2), nbr_out(c - 1), wsem[2 + b2]).wait()
                pltpu.async_copy(emb_hbm.at[idx_v[b2]], nrows_v[b2],
                                 gsem[2 + b2])
        for c in (n_nbr - 2, n_nbr - 1):
            pltpu.make_async_copy(
                nbr_src(c & 1), nbr_out(c), wsem[2 + (c & 1)]).wait()

        for c in range(n_zq):
            b = c & 1
            pltpu.make_async_copy(
                emb_hbm.at[bmu_v.at[pl.ds(c * zq_chunk, zq_chunk)]],
                zrows_v[b], gsem[b]).wait()
            pltpu.async_copy(zrows_v[b], zq_out(c), wsem[b])
            if c + 2 < n_zq:
                pltpu.make_async_copy(zrows_v[b], zq_out(c), wsem[b]).wait()
                pltpu.async_copy(
                    emb_hbm.at[bmu_v.at[pl.ds((c + 2) * zq_chunk, zq_chunk)]],
                    zrows_v[b], gsem[b])
        for c in (n_zq - 2, n_zq - 1):
            pltpu.make_async_copy(
                zrows_v[c & 1], zq_out(c), wsem[c & 1]).wait()

    return k(embeddings, bmu)


def kernel(z_e_sample, embeddings, alpha_som):
    af = jnp.asarray(alpha_som, jnp.float32).reshape(1, 1)
    z_dist, bmu = _tc_dist_call(z_e_sample, embeddings)
    z_q, nbr8 = _sc_gather(embeddings, bmu)
    q = _tc_q_call(af, z_e_sample, embeddings)
    z_q_neighbors = nbr8[:, :5, :]
    return (z_dist, bmu, z_q, z_q_neighbors, q)


# speedup vs baseline: 1.0501x; 1.0003x over previous
import functools

import jax
import jax.numpy as jnp
from jax import lax
from jax.experimental import pallas as pl
from jax.experimental.pallas import tpu as pltpu
from jax.experimental.pallas import tpu_sc as plsc

SOM_H, SOM_W = 64, 64
N_NODES = SOM_H * SOM_W
LATENT = 256
N_ROWS = 8192
TC_BLOCK = 512

_EPS_F32 = 1.1920929e-07


def _dist_body(z_ref, e_ref, dist_ref, bmu_ref, e2_ref):
    zb = z_ref[...]

    @pl.when(pl.program_id(0) == 0)
    def _():
        eb = e_ref[...]
        e2_ref[...] = jnp.sum(eb * eb, axis=1)[None, :]

    ones_d = jnp.ones((LATENT,), jnp.float32)
    z2 = lax.dot_general(zb * zb, ones_d, (((1,), (0,)), ((), ())),
                         preferred_element_type=jnp.float32)[:, None]
    dot = lax.dot_general(zb, e_ref[...], (((1,), (1,)), ((), ())),
                          preferred_element_type=jnp.float32)
    d = z2 + e2_ref[...] - 2.0 * dot
    d = jnp.maximum(d, 0.0)
    dist_ref[...] = d
    bmu_ref[...] = jnp.argmin(d, axis=1).astype(jnp.int32)


def _q_body(alpha_ref, z_ref, e_ref, q_ref, e2_ref):
    zb = z_ref[...]

    @pl.when(pl.program_id(0) == 0)
    def _():
        eb = e_ref[...]
        e2_ref[...] = jnp.sum(eb * eb, axis=1)[None, :]

    ones_d = jnp.ones((LATENT,), jnp.float32)
    ones_k = jnp.ones((N_NODES,), jnp.float32)
    z2 = lax.dot_general(zb * zb, ones_d, (((1,), (0,)), ((), ())),
                         preferred_element_type=jnp.float32)[:, None]
    dot = lax.dot_general(zb, e_ref[...], (((1,), (1,)), ((), ())),
                          preferred_element_type=jnp.float32)
    d = z2 + e2_ref[...] - 2.0 * dot
    d = jnp.maximum(d, 0.0)

    af = alpha_ref[0, 0]
    ex = (af + 1.0) * 0.5
    ia = 1.0 / af

    def _finish(qn):
        s = lax.dot_general(qn, ones_k, (((1,), (0,)), ((), ())),
                            preferred_element_type=jnp.float32)[:, None]
        q_ref[...] = qn * (1.0 / s) + _EPS_F32

    @pl.when(ex == 1.0)
    def _():
        _finish(1.0 / (1.0 + d * ia))

    @pl.when(ex != 1.0)
    def _():
        _finish(jnp.exp(jnp.log(1.0 / (1.0 + d * ia)) * ex))


def _tc_dist_call(z, e):
    grid = (N_ROWS // TC_BLOCK,)
    return pl.pallas_call(
        _dist_body,
        grid=grid,
        in_specs=[
            pl.BlockSpec((TC_BLOCK, LATENT), lambda i: (i, 0)),
            pl.BlockSpec((N_NODES, LATENT), lambda i: (0, 0)),
        ],
        out_specs=[
            pl.BlockSpec((TC_BLOCK, N_NODES), lambda i: (i, 0)),
            pl.BlockSpec((TC_BLOCK,), lambda i: (i,)),
        ],
        out_shape=[
            jax.ShapeDtypeStruct((N_ROWS, N_NODES), jnp.float32),
            jax.ShapeDtypeStruct((N_ROWS,), jnp.int32),
        ],
        scratch_shapes=[pltpu.VMEM((1, N_NODES), jnp.float32)],
    )(z, e)


def _tc_q_call(af, z, e):
    grid = (N_ROWS // TC_BLOCK,)
    return pl.pallas_call(
        _q_body,
        grid=grid,
        in_specs=[
            pl.BlockSpec(memory_space=pltpu.SMEM),
            pl.BlockSpec((TC_BLOCK, LATENT), lambda i: (i, 0)),
            pl.BlockSpec((N_NODES, LATENT), lambda i: (0, 0)),
        ],
        out_specs=pl.BlockSpec((TC_BLOCK, N_NODES), lambda i: (i, 0)),
        out_shape=jax.ShapeDtypeStruct((N_ROWS, N_NODES), jnp.float32),
        scratch_shapes=[pltpu.VMEM((1, N_NODES), jnp.float32)],
    )(af, z, e)


def _sc_gather(embeddings, bmu):
    info = plsc.get_sparse_core_info()
    nc, ns = info.num_cores, info.num_subcores
    nw = nc * ns
    rows_w = N_ROWS // nw
    zq_chunk = 64
    nbr_rows = 16
    n_zq = rows_w // zq_chunk
    n_nbr = rows_w // nbr_rows

    mesh = plsc.VectorSubcoreMesh(core_axis_name="c", subcore_axis_name="s")

    @functools.partial(
        pl.kernel,
        mesh=mesh,
        out_type=[
            jax.ShapeDtypeStruct((N_ROWS, LATENT), jnp.float32),
            jax.ShapeDtypeStruct((N_ROWS, 8, LATENT), jnp.float32),
        ],
        scratch_types=[
            pltpu.VMEM((rows_w + 8,), jnp.int32),
            [pltpu.VMEM((nbr_rows * 8,), jnp.int32) for _ in range(2)],
            [pltpu.VMEM((zq_chunk, LATENT), jnp.float32) for _ in range(2)],
            [pltpu.VMEM((nbr_rows * 8, LATENT), jnp.float32)
             for _ in range(2)],
            [pltpu.SemaphoreType.DMA for _ in range(4)],
            [pltpu.SemaphoreType.DMA for _ in range(4)],
        ],
    )
    def k(emb_hbm, bmu_hbm, zq_hbm, nbr_hbm, bmu_v, idx_v, zrows_v,
          nrows_v, gsem, wsem):
        wid = lax.axis_index("s") * nc + lax.axis_index("c")
        base = wid * rows_w
        pltpu.sync_copy(bmu_hbm.at[pl.ds(base, rows_w)],
                        bmu_v.at[pl.ds(0, rows_w)])

        lane = lax.iota(jnp.int32, 16)
        dnums = lax.GatherDimensionNumbers(
            offset_dims=(), collapsed_slice_dims=(0,), start_index_map=(0,))

        def nbr_idx(c, idx_ref):
            v = bmu_v[pl.ds(c * nbr_rows, 16)]
            for t in range(nbr_rows * 8 // 16):
                p = t * 16 + lane
                i_rel = lax.shift_right_logical(p, 3)
                j_rel = p & 7
                vi = lax.gather(
                    v, i_rel[:, None], dnums, (1,),
                    mode=lax.GatherScatterMode.PROMISE_IN_BOUNDS)
                k1 = lax.shift_right_logical(vi, 6)
                k2 = vi & 63
                dk1 = (jnp.where(j_rel == 1, -1, 0)
                       + jnp.where(j_rel == 2, 1, 0))
                dk2 = (jnp.where(j_rel == 3, 1, 0)
                       + jnp.where(j_rel == 4, -1, 0))
                idx_ref[pl.ds(t * 16, 16)] = (
                    lax.shift_left((k1 + dk1 + 64) & 63, 6)
                    | ((k2 + dk2 + 64) & 63))

        def zq_out(c):
            return zq_hbm.at[pl.ds(base + c * zq_chunk, zq_chunk)]

        def nbr_out(c):
            return nbr_hbm.at[pl.ds(base + c * nbr_rows, nbr_rows)]

        def nbr_src(b):
            return nrows_v[b].reshape(nbr_rows, 8, LATENT)

        for c in range(2):
            pltpu.async_copy(
                emb_hbm.at[bmu_v.at[pl.ds(c * zq_chunk, zq_chunk)]],
                zrows_v[c], gsem[c])

        nbr_idx(0, idx_v[0])
        pltpu.async_copy(emb_hbm.at[idx_v[0]], nrows_v[0], gsem[2])
        for c in range(n_nbr):
            b = c & 1
            pltpu.make_async_copy(
                emb_hbm.at[idx_v[b]], nrows_v[b], gsem[2 + b]).wait()
            pltpu.async_copy(nbr_src(b), nbr_out(c), wsem[2 + b])
            if c + 1 < n_nbr:
                b2 = (c + 1) & 1
                nbr_idx(c + 1, idx_v[b2])
                if c >= 1:
                    pltpu.make_async_copy(
                        nbr_src(b2), nbr_out(c - 1), wsem[2 + b2]).wait()
                pltpu.async_copy(emb_hbm.at[idx_v[b2]], nrows_v[b2],
                                 gsem[2 + b2])
        for c in (n_nbr - 2, n_nbr - 1):
            pltpu.make_async_copy(
                nbr_src(c & 1), nbr_out(c), wsem[2 + (c & 1)]).wait()

        for c in range(n_zq):
            b = c & 1
            pltpu.make_async_copy(
                emb_hbm.at[bmu_v.at[pl.ds(c * zq_chunk, zq_chunk)]],
                zrows_v[b], gsem[b]).wait()
            pltpu.async_copy(zrows_v[b], zq_out(c), wsem[b])
            if c + 2 < n_zq:
                pltpu.make_async_copy(zrows_v[b], zq_out(c), wsem[b]).wait()
                pltpu.async_copy(
                    emb_hbm.at[bmu_v.at[pl.ds((c + 2) * zq_chunk, zq_chunk)]],
                    zrows_v[b], gsem[b])
        for c in (n_zq - 2, n_zq - 1):
            pltpu.make_async_copy(
                zrows_v[c & 1], zq_out(c), wsem[c & 1]).wait()

    return k(embeddings, bmu)


def kernel(z_e_sample, embeddings, alpha_som):
    af = jnp.asarray(alpha_som, jnp.float32).reshape(1, 1)
    z_dist, bmu = _tc_dist_call(z_e_sample, embeddings)
    q = _tc_q_call(af, z_e_sample, embeddings)
    z_q, nbr8 = _sc_gather(embeddings, bmu)
    z_q_neighbors = nbr8[:, :5, :]
    return (z_dist, bmu, z_q, z_q_neighbors, q)
